# baseline (speedup 1.0000x reference)
import functools

import jax
import jax.numpy as jnp
from jax.experimental import pallas as pl
from jax.experimental.pallas import tpu as pltpu

B, T, C, K = 8, 1024, 128, 3
MIN_P, MAX_P = 16, 64
P_MAX = MAX_P
CYC_MAX = (T + MIN_P - 1) // MIN_P
KER = 9
GSIZE = 4
NEG = float(jnp.finfo(jnp.float32).min) / 8.0
KC = K * CYC_MAX
RTAIL = T - 128


def _dsd_body(x_ref, dwT_ref, pw_ref, gng_ref, gnb_ref, gate_ref, rg_ref,
              out_ref):
    xb = x_ref[0]

    xp = jnp.concatenate([xb, xb[:MAX_P, :]], axis=0)
    BLK = 64
    gaccs = [jnp.zeros((BLK, BLK + MAX_P), jnp.float32) for _ in range(4)]
    for q in range(T // BLK):
        gaccs[q % 4] = gaccs[q % 4] + jax.lax.dot_general(
            xb[q * BLK:(q + 1) * BLK, :],
            xp[q * BLK:q * BLK + BLK + MAX_P, :],
            (((1,), (1,)), ((), ())), preferred_element_type=jnp.float32)
    gsum = (gaccs[0] + gaccs[1]) + (gaccs[2] + gaccs[3])
    srow = jax.lax.broadcasted_iota(jnp.int32, (BLK, BLK + MAX_P), 0)
    for bit in (1, 2, 4, 8, 16, 32):
        rolled = jnp.roll(gsum, -bit, axis=1)
        gsum = jnp.where((srow & bit) != 0, rolled, gsum)
    r_row = jnp.sum(gsum, axis=0, keepdims=True) * (1.0 / C)

    lanef = jax.lax.broadcasted_iota(
        jnp.int32, (1, BLK + MAX_P), 1).astype(jnp.float32)
    rm = jnp.where((lanef >= jnp.float32(MIN_P)) &
                   (lanef <= jnp.float32(MAX_P)), r_row, NEG)
    top_v, top_p = [], []
    for _ in range(K):
        m = jnp.max(rm)
        pidx = jnp.min(jnp.where(rm == m, lanef, jnp.float32(1e9)))
        top_v.append(m)
        top_p.append(pidx)
        rm = jnp.where(lanef == pidx, NEG, rm)
    exps = [jnp.exp(v - top_v[0]) for v in top_v]
    esum = exps[0] + exps[1] + exps[2]
    ws = [e / esum for e in exps]
    recips = [1.0 / p for p in top_p]
    ncycs = [jnp.floor((jnp.float32(T) - 0.5) * r) + 1.0 for r in recips]
    tpads = [n * p for n, p in zip(ncycs, top_p)]

    gng = gng_ref[...]
    gnb = gnb_ref[...]
    rg = rg_ref[...]
    ci = jax.lax.broadcasted_iota(jnp.int32, (C, C), 0).astype(jnp.float32)
    cj = jax.lax.broadcasted_iota(jnp.int32, (C, C), 1).astype(jnp.float32)
    gmat = (jnp.floor(ci * (1.0 / GSIZE)) ==
            jnp.floor(cj * (1.0 / GSIZE))).astype(jnp.float32)

    recip_col = jnp.concatenate(
        [jnp.full((CYC_MAX, 1), r, jnp.float32) for r in recips], axis=0)
    tpad_col = jnp.concatenate(
        [jnp.full((CYC_MAX, 1), t, jnp.float32) for t in tpads], axis=0)

    tW = jax.lax.broadcasted_iota(jnp.int32, (KC, T), 1).astype(jnp.float32)
    cycW = (jax.lax.broadcasted_iota(jnp.int32, (KC, T), 0) &
            (CYC_MAX - 1)).astype(jnp.float32)
    fd1 = jnp.floor((tW + 0.5) * recip_col)
    w1 = jnp.where((fd1 == cycW) & (tW < tpad_col), 1.0, 0.0)
    jW = jax.lax.broadcasted_iota(jnp.int32, (KC, 128), 1).astype(jnp.float32)
    cycWs = (jax.lax.broadcasted_iota(jnp.int32, (KC, 128), 0) &
             (CYC_MAX - 1)).astype(jnp.float32)
    s2 = jnp.float32(2 * (T - 1) - RTAIL) - jW
    fd2 = jnp.floor((s2 + 0.5) * recip_col)
    w2 = jnp.where((fd2 == cycWs) & (s2 >= jnp.float32(T)) & (s2 < tpad_col),
                   1.0, 0.0)
    ustack = (jax.lax.dot(w1, xb, preferred_element_type=jnp.float32) +
              jax.lax.dot(w2, xb[RTAIL:, :],
                          preferred_element_type=jnp.float32)) * (1.0 / P_MAX)

    ub8 = jnp.concatenate(
        [jnp.sum(ustack[k * CYC_MAX:(k + 1) * CYC_MAX, :], axis=0,
                 keepdims=True) * (1.0 / CYC_MAX) for k in range(K)] +
        [jnp.zeros((8 - K, C), jnp.float32)], axis=0)
    gg = jax.lax.dot_general(ub8, gate_ref[...], (((1,), (1,)), ((), ())),
                             preferred_element_type=jnp.float32)
    grows = [1.0 / (1.0 + jnp.exp(-gg[k:k + 1, :])) for k in range(K)]

    z4 = jnp.zeros((KER // 2, C), jnp.float32)
    xc1s = []
    for k in range(K):
        up = jnp.concatenate(
            [z4, ustack[k * CYC_MAX:(k + 1) * CYC_MAX, :], z4], axis=0)
        xc1 = dwT_ref[0:1, :] * up[0:CYC_MAX, :]
        for j in range(1, KER):
            xc1 = xc1 + dwT_ref[j:j + 1, :] * up[j:j + CYC_MAX, :]
        xc1s.append(xc1)
    xc1 = jnp.concatenate(xc1s, axis=0)

    xc2 = jax.lax.dot_general(xc1, pw_ref[...], (((1,), (1,)), ((), ())),
                              preferred_element_type=jnp.float32)
    m1 = jax.lax.dot(xc2, gmat, preferred_element_type=jnp.float32)
    m2 = jax.lax.dot(xc2 * xc2, gmat, preferred_element_type=jnp.float32)
    denom = 1.0 / (GSIZE * CYC_MAX)
    mks = []
    for k in range(K):
        sl = slice(k * CYC_MAX, (k + 1) * CYC_MAX)
        mu = jnp.sum(m1[sl, :], axis=0, keepdims=True) * denom
        var = jnp.sum(m2[sl, :], axis=0, keepdims=True) * denom - mu * mu
        xn = (xc2[sl, :] - mu) * jax.lax.rsqrt(var + 1e-5) * gng + gnb
        xg = 0.5 * xn * (1.0 + jax.lax.erf(xn * 0.7071067811865476))
        mks.append(xg * (grows[k] * rg * ws[k]))
    mks = jnp.concatenate(mks, axis=0)

    acc = jax.lax.dot_general(w1, mks, (((0,), (0,)), ((), ())),
                              preferred_element_type=jnp.float32)

    wsum = (ws[0] + ws[1]) + ws[2]
    out_ref[0] = xb * (1.0 + wsum) + acc


@jax.jit
def kernel(x, dw_w, pw_w, gn_g, gn_b, gate_w, res_g):
    dwT = jnp.transpose(dw_w[:, 0, :], (1, 0))
    gng = gn_g.reshape(1, C)
    gnb = gn_b.reshape(1, C)
    rg = res_g.reshape(1, C)
    grid_spec = pl.GridSpec(
        grid=(B,),
        in_specs=[
            pl.BlockSpec((1, T, C), lambda b: (b, 0, 0)),
            pl.BlockSpec((KER, C), lambda b: (0, 0)),
            pl.BlockSpec((C, C), lambda b: (0, 0)),
            pl.BlockSpec((1, C), lambda b: (0, 0)),
            pl.BlockSpec((1, C), lambda b: (0, 0)),
            pl.BlockSpec((C, C), lambda b: (0, 0)),
            pl.BlockSpec((1, C), lambda b: (0, 0)),
        ],
        out_specs=pl.BlockSpec((1, T, C), lambda b: (b, 0, 0)),
    )
    return pl.pallas_call(
        _dsd_body,
        grid_spec=grid_spec,
        out_shape=jax.ShapeDtypeStruct((B, T, C), jnp.float32),
    )(x, dwT, pw_w, gng, gnb, gate_w, rg)


# speedup vs baseline: 1.2547x; 1.2547x over previous
import functools

import jax
import jax.numpy as jnp
from jax.experimental import pallas as pl
from jax.experimental.pallas import tpu as pltpu

B, T, C, K = 8, 1024, 128, 3
MIN_P, MAX_P = 16, 64
P_MAX = MAX_P
CYC_MAX = (T + MIN_P - 1) // MIN_P
KER = 9
GSIZE = 4
NEG = float(jnp.finfo(jnp.float32).min) / 8.0
KC = K * CYC_MAX
RTAIL = T - 128


def _dsd_body(x_ref, dwT_ref, pw_ref, gng_ref, gnb_ref, gate_ref, rg_ref,
              out_ref):
    for i in range(x_ref.shape[0]):
        _dsd_one(x_ref[i], dwT_ref, pw_ref, gng_ref, gnb_ref, gate_ref,
                 rg_ref, out_ref, i)


def _dsd_one(xb, dwT_ref, pw_ref, gng_ref, gnb_ref, gate_ref, rg_ref,
             out_ref, i):
    xp = jnp.concatenate([xb, xb[:MAX_P, :]], axis=0)
    BLK = 64
    gaccs = [jnp.zeros((BLK, BLK + MAX_P), jnp.float32) for _ in range(4)]
    for q in range(T // BLK):
        gaccs[q % 4] = gaccs[q % 4] + jax.lax.dot_general(
            xb[q * BLK:(q + 1) * BLK, :],
            xp[q * BLK:q * BLK + BLK + MAX_P, :],
            (((1,), (1,)), ((), ())), preferred_element_type=jnp.float32)
    gsum = (gaccs[0] + gaccs[1]) + (gaccs[2] + gaccs[3])
    diag = (jax.lax.broadcasted_iota(jnp.int32, (BLK, BLK + MAX_P), 1) -
            jax.lax.broadcasted_iota(jnp.int32, (BLK, BLK + MAX_P), 0))
    lag_vals = []
    for lag in range(MIN_P, MAX_P + 1):
        lag_vals.append(
            jnp.sum(jnp.where(diag == lag, gsum, 0.0)) * (1.0 / C))

    lags = list(range(MIN_P, MAX_P + 1))
    vals = list(lag_vals)
    top_v, top_p = [], []
    for _ in range(K):
        m = vals[0]
        for v in vals[1:]:
            m = jnp.maximum(m, v)
        idx = jnp.float32(lags[-1])
        for lag, v in zip(reversed(lags), reversed(vals)):
            idx = jnp.where(v == m, jnp.float32(lag), idx)
        top_v.append(m)
        top_p.append(idx)
        vals = [jnp.where(jnp.float32(lag) == idx, NEG, v)
                for lag, v in zip(lags, vals)]
    exps = [jnp.exp(v - top_v[0]) for v in top_v]
    esum = exps[0] + exps[1] + exps[2]
    ws = [e / esum for e in exps]
    recips = [1.0 / p for p in top_p]
    ncycs = [jnp.floor((jnp.float32(T) - 0.5) * r) + 1.0 for r in recips]
    tpads = [n * p for n, p in zip(ncycs, top_p)]

    gng = gng_ref[...]
    gnb = gnb_ref[...]
    rg = rg_ref[...]
    ci = jax.lax.broadcasted_iota(jnp.int32, (C, C), 0).astype(jnp.float32)
    cj = jax.lax.broadcasted_iota(jnp.int32, (C, C), 1).astype(jnp.float32)
    gmat = (jnp.floor(ci * (1.0 / GSIZE)) ==
            jnp.floor(cj * (1.0 / GSIZE))).astype(jnp.float32)

    recip_col = jnp.concatenate(
        [jnp.full((CYC_MAX, 1), r, jnp.float32) for r in recips], axis=0)
    tpad_col = jnp.concatenate(
        [jnp.full((CYC_MAX, 1), t, jnp.float32) for t in tpads], axis=0)

    tW = jax.lax.broadcasted_iota(jnp.int32, (KC, T), 1).astype(jnp.float32)
    cycW = (jax.lax.broadcasted_iota(jnp.int32, (KC, T), 0) &
            (CYC_MAX - 1)).astype(jnp.float32)
    fd1 = jnp.floor((tW + 0.5) * recip_col)
    w1 = jnp.where((fd1 == cycW) & (tW < tpad_col), 1.0, 0.0)
    jW = jax.lax.broadcasted_iota(jnp.int32, (KC, 128), 1).astype(jnp.float32)
    cycWs = (jax.lax.broadcasted_iota(jnp.int32, (KC, 128), 0) &
             (CYC_MAX - 1)).astype(jnp.float32)
    s2 = jnp.float32(2 * (T - 1) - RTAIL) - jW
    fd2 = jnp.floor((s2 + 0.5) * recip_col)
    w2 = jnp.where((fd2 == cycWs) & (s2 >= jnp.float32(T)) & (s2 < tpad_col),
                   1.0, 0.0)
    ustack = (jax.lax.dot(w1, xb, preferred_element_type=jnp.float32) +
              jax.lax.dot(w2, xb[RTAIL:, :],
                          preferred_element_type=jnp.float32)) * (1.0 / P_MAX)

    ub8 = jnp.concatenate(
        [jnp.sum(ustack[k * CYC_MAX:(k + 1) * CYC_MAX, :], axis=0,
                 keepdims=True) * (1.0 / CYC_MAX) for k in range(K)] +
        [jnp.zeros((8 - K, C), jnp.float32)], axis=0)
    gg = jax.lax.dot_general(ub8, gate_ref[...], (((1,), (1,)), ((), ())),
                             preferred_element_type=jnp.float32)
    grows = [1.0 / (1.0 + jnp.exp(-gg[k:k + 1, :])) for k in range(K)]

    z4 = jnp.zeros((KER // 2, C), jnp.float32)
    xc1s = []
    for k in range(K):
        up = jnp.concatenate(
            [z4, ustack[k * CYC_MAX:(k + 1) * CYC_MAX, :], z4], axis=0)
        xc1 = dwT_ref[0:1, :] * up[0:CYC_MAX, :]
        for j in range(1, KER):
            xc1 = xc1 + dwT_ref[j:j + 1, :] * up[j:j + CYC_MAX, :]
        xc1s.append(xc1)
    xc1 = jnp.concatenate(xc1s, axis=0)

    xc2 = jax.lax.dot_general(xc1, pw_ref[...], (((1,), (1,)), ((), ())),
                              preferred_element_type=jnp.float32)
    m1 = jax.lax.dot(xc2, gmat, preferred_element_type=jnp.float32)
    m2 = jax.lax.dot(xc2 * xc2, gmat, preferred_element_type=jnp.float32)
    denom = 1.0 / (GSIZE * CYC_MAX)
    mks = []
    for k in range(K):
        sl = slice(k * CYC_MAX, (k + 1) * CYC_MAX)
        mu = jnp.sum(m1[sl, :], axis=0, keepdims=True) * denom
        var = jnp.sum(m2[sl, :], axis=0, keepdims=True) * denom - mu * mu
        xn = (xc2[sl, :] - mu) * jax.lax.rsqrt(var + 1e-5) * gng + gnb
        xg = 0.5 * xn * (1.0 + jax.lax.erf(xn * 0.7071067811865476))
        mks.append(xg * (grows[k] * rg * ws[k]))
    mks = jnp.concatenate(mks, axis=0)

    tE = jax.lax.broadcasted_iota(jnp.int32, (T, KC), 0).astype(jnp.float32)
    cycE = (jax.lax.broadcasted_iota(jnp.int32, (T, KC), 1) &
            (CYC_MAX - 1)).astype(jnp.float32)
    recip_row = jnp.concatenate(
        [jnp.full((1, CYC_MAX), r, jnp.float32) for r in recips], axis=1)
    fde = jnp.floor((tE + 0.5) * recip_row)
    emat = jnp.where(fde == cycE, 1.0, 0.0)
    acc = jax.lax.dot(emat, mks, preferred_element_type=jnp.float32)

    wsum = (ws[0] + ws[1]) + ws[2]
    out_ref[i] = xb * (1.0 + wsum) + acc


@jax.jit
def kernel(x, dw_w, pw_w, gn_g, gn_b, gate_w, res_g):
    dwT = jnp.transpose(dw_w[:, 0, :], (1, 0))
    gng = gn_g.reshape(1, C)
    gnb = gn_b.reshape(1, C)
    rg = res_g.reshape(1, C)
    SPB = 2
    grid_spec = pl.GridSpec(
        grid=(B // SPB,),
        in_specs=[
            pl.BlockSpec((SPB, T, C), lambda b: (b, 0, 0)),
            pl.BlockSpec((KER, C), lambda b: (0, 0)),
            pl.BlockSpec((C, C), lambda b: (0, 0)),
            pl.BlockSpec((1, C), lambda b: (0, 0)),
            pl.BlockSpec((1, C), lambda b: (0, 0)),
            pl.BlockSpec((C, C), lambda b: (0, 0)),
            pl.BlockSpec((1, C), lambda b: (0, 0)),
        ],
        out_specs=pl.BlockSpec((SPB, T, C), lambda b: (b, 0, 0)),
    )
    return pl.pallas_call(
        _dsd_body,
        grid_spec=grid_spec,
        out_shape=jax.ShapeDtypeStruct((B, T, C), jnp.float32),
    )(x, dwT, pw_w, gng, gnb, gate_w, rg)
